# baseline (speedup 1.0000x reference)
import functools

import jax
import jax.numpy as jnp
from jax import lax
from jax.experimental import pallas as pl
from jax.experimental.pallas import tpu as pltpu
from jax.experimental.pallas import tpu_sc as plsc

N = 10000
NP = 10240
CIN = 6
C = 128
K = 16
DEPTH = 6

BQ = 128
RC = 512
NCH = NP // RC
NW = 32
RPW = NP // NW
CH = 8

F32 = jnp.float32


NG = NP // 128
GK = 4


def _knn_body(xtp_ref, xqt_ref, idx_ref):
    xtp = xtp_ref[...]
    xqt = xqt_ref[...]
    prod = jnp.dot(xtp, xqt, preferred_element_type=F32)
    sqj = jnp.sum(xtp * xtp, axis=1, keepdims=True)
    sqq = jnp.sum(xqt * xqt, axis=0, keepdims=True)
    rowi = lax.broadcasted_iota(jnp.int32, (NP, 1), 0)
    d = (sqq + sqj) - 2.0 * prod
    d = jnp.where(rowi < N, d, jnp.float32(jnp.inf))

    riota = lax.broadcasted_iota(jnp.int32, (128, BQ), 0)
    gvs = [[] for _ in range(GK)]
    gas = [[] for _ in range(GK)]
    for g in range(NG):
        chunk = d[g * 128:(g + 1) * 128]
        for t in range(GK):
            m = jnp.min(chunk, axis=0, keepdims=True)
            a = jnp.min(jnp.where(chunk == m, riota, 128), axis=0,
                        keepdims=True)
            gvs[t].append(m)
            gas[t].append(a)
            if t + 1 < GK:
                chunk = jnp.where(riota == a, jnp.float32(jnp.inf), chunk)
    gv = [jnp.concatenate(v, axis=0) for v in gvs]
    ga = [jnp.concatenate(a, axis=0) for a in gas]

    giota = lax.broadcasted_iota(jnp.int32, (NG, BQ), 0)
    inf = jnp.float32(jnp.inf)
    hv, ha, hc = gv[0], ga[0], jnp.zeros((NG, BQ), jnp.int32)
    picks = []
    for _ in range(K):
        m = jnp.min(hv, axis=0, keepdims=True)
        s = jnp.min(jnp.where(hv == m, giota, NG), axis=0, keepdims=True)
        hit = giota == s
        j_in = jnp.min(jnp.where(hit, ha, NP), axis=0, keepdims=True)
        picks.append(s * 128 + j_in)
        hc = hc + hit.astype(jnp.int32)
        nv, na = inf, jnp.int32(NP)
        for t in range(GK - 1, 0, -1):
            sel = hc == t
            nv = jnp.where(sel, gv[t], nv)
            na = jnp.where(sel, ga[t], na)
        hv = jnp.where(hit, nv, hv)
        ha = jnp.where(hit, na, ha)
    idx_ref[...] = jnp.concatenate(picks, axis=0)


def _knn(xtp8):
    xtt = xtp8.T
    return pl.pallas_call(
        _knn_body,
        grid=(NP // BQ,),
        in_specs=[
            pl.BlockSpec((NP, 8), lambda i: (0, 0)),
            pl.BlockSpec((8, BQ), lambda i: (0, i)),
        ],
        out_specs=pl.BlockSpec((K, BQ), lambda i: (0, i)),
        out_shape=jax.ShapeDtypeStruct((K, NP), jnp.int32),
    )(xtp8, xtt)


NCHK = RPW // CH


def _gmax_body(D, feat_hbm, idx_hbm, out_hbm,
               idxall, rows0, rows1, outv0, outv1,
               gsem0, gsem1, osem0, osem1):
    wid = lax.axis_index("s") * 2 + lax.axis_index("c")
    row0 = wid * RPW
    rows = (rows0, rows1)
    outv = (outv0, outv1)
    gsem = (gsem0, gsem1)
    osem = (osem0, osem1)

    pltpu.sync_copy(idx_hbm.at[pl.ds(pl.multiple_of(row0 * K, 128), RPW * K)],
                    idxall)

    def fetch(i, buf):
        idxs = idxall.at[pl.ds(pl.multiple_of(i * CH * K, 128), CH * K)]
        pltpu.async_copy(feat_hbm.at[idxs], rows[buf], gsem[buf])

    def consume(i, buf, drain):
        idxs = idxall.at[pl.ds(pl.multiple_of(i * CH * K, 128), CH * K)]
        pltpu.make_async_copy(feat_hbm.at[idxs], rows[buf], gsem[buf]).wait()
        base = row0 + i * CH
        dst = out_hbm.at[pl.ds(pl.multiple_of(base, 8), CH)]

        @pl.when(drain)
        def _():
            pltpu.make_async_copy(outv[buf], dst, osem[buf]).wait()

        r = rows[buf]
        for dd in range(CH):
            for lc in range(D // 16):
                sl = pl.ds(lc * 16, 16)
                vals = [r[dd * K + j, sl] for j in range(K)]
                while len(vals) > 1:
                    vals = [jnp.maximum(vals[t], vals[t + 1])
                            for t in range(0, len(vals), 2)]
                outv[buf][dd, sl] = vals[0]
        pltpu.async_copy(outv[buf], dst, osem[buf])

    fetch(0, 0)

    def pair(i2, carry):
        e = i2 * 2
        fetch(e + 1, 1)
        consume(e, 0, i2 > 0)

        @pl.when(e + 2 < NCHK)
        def _():
            fetch(e + 2, 0)

        consume(e + 1, 1, i2 > 0)
        return carry

    lax.fori_loop(0, NCHK // 2, pair, 0)
    tail = out_hbm.at[pl.ds(pl.multiple_of(row0, 8), CH)]
    pltpu.make_async_copy(outv[0], tail, osem[0]).wait()
    pltpu.make_async_copy(outv[1], tail, osem[1]).wait()


def _gmax(feat, idxf, D):
    mesh = plsc.VectorSubcoreMesh(core_axis_name="c", subcore_axis_name="s")
    return pl.kernel(
        functools.partial(_gmax_body, D),
        out_type=jax.ShapeDtypeStruct((NP, D), F32),
        mesh=mesh,
        scratch_types=[
            pltpu.VMEM((RPW * K,), jnp.int32),
            pltpu.VMEM((CH * K, D), F32),
            pltpu.VMEM((CH * K, D), F32),
            pltpu.VMEM((CH, D), F32),
            pltpu.VMEM((CH, D), F32),
            pltpu.SemaphoreType.DMA,
            pltpu.SemaphoreType.DMA,
            pltpu.SemaphoreType.DMA,
            pltpu.SemaphoreType.DMA,
        ],
    )(feat, idxf)


def _conv_in_body(f_ref, a_ref, w1_ref, w2_ref, b_ref, o_ref):
    o_ref[...] = (jnp.dot(f_ref[...], w1_ref[...], preferred_element_type=F32)
                  + jnp.dot(a_ref[...], w2_ref[...], preferred_element_type=F32)
                  + b_ref[...])


def _conv_in(xtp8, agg0, w1, w2, b):
    return pl.pallas_call(
        _conv_in_body,
        grid=(NCH,),
        in_specs=[
            pl.BlockSpec((RC, 8), lambda c: (c, 0)),
            pl.BlockSpec((RC, C), lambda c: (c, 0)),
            pl.BlockSpec((8, C), lambda c: (0, 0)),
            pl.BlockSpec((C, C), lambda c: (0, 0)),
            pl.BlockSpec((1, C), lambda c: (0, 0)),
        ],
        out_specs=pl.BlockSpec((RC, C), lambda c: (c, 0)),
        out_shape=jax.ShapeDtypeStruct((NP, C), F32),
    )(xtp8, agg0, w1, w2, b)


def _conv_norm_body(with_res, *refs):
    if with_res:
        f_ref, a_ref, w1_ref, w2_ref, b_ref, g_ref, bt_ref, r_ref, o_ref, acc = refs
    else:
        f_ref, a_ref, w1_ref, w2_ref, b_ref, g_ref, bt_ref, o_ref, acc = refs
    p = pl.program_id(0)
    c = pl.program_id(1)
    y = (jnp.dot(f_ref[...], w1_ref[...], preferred_element_type=F32)
         + jnp.dot(a_ref[...], w2_ref[...], preferred_element_type=F32)
         + b_ref[...])
    rows = lax.broadcasted_iota(jnp.int32, (RC, 1), 0) + c * RC
    msk = rows < N

    @pl.when(p == 0)
    def _():
        @pl.when(c == 0)
        def _():
            acc[...] = jnp.zeros_like(acc)
        ym = jnp.where(msk, y, 0.0)
        acc[0:1, :] = acc[0:1, :] + jnp.sum(ym, axis=0, keepdims=True)
        acc[1:2, :] = acc[1:2, :] + jnp.sum(ym * ym, axis=0, keepdims=True)

    @pl.when(p == 1)
    def _():
        m = acc[0:1, :] * (1.0 / N)
        v = acc[1:2, :] * (1.0 / N) - m * m
        s1 = 1.0 / jnp.sqrt(v + 1e-3)
        v2 = v * s1 * s1
        s = g_ref[...] * s1 / jnp.sqrt(v2 + 1e-5)
        out = jnp.maximum((y - m) * s + bt_ref[...], 0.0)
        if with_res:
            out = out + r_ref[...]
        o_ref[...] = out


def _conv_norm(feat, agg, w1, w2, b, g, bt, res):
    with_res = res is not None
    ops = (feat, agg, w1, w2, b, g, bt) + ((res,) if with_res else ())
    chunked = pl.BlockSpec((RC, C), lambda p, c: (c, 0))
    whole = lambda shape: pl.BlockSpec(shape, lambda p, c: (0, 0))
    in_specs = [chunked, chunked, whole((C, C)), whole((C, C)), whole((1, C)),
                whole((1, C)), whole((1, C))] + ([chunked] if with_res else [])
    return pl.pallas_call(
        functools.partial(_conv_norm_body, with_res),
        grid=(2, NCH),
        in_specs=in_specs,
        out_specs=chunked,
        out_shape=jax.ShapeDtypeStruct((NP, C), F32),
        scratch_shapes=[pltpu.VMEM((8, C), F32)],
    )(*ops)


def kernel(x, W0, b0, Wa, ba, ga, bta, Wb, bb, gb, btb):
    xt = x[0].T
    xtp128 = jnp.zeros((NP, C), F32).at[:N, :CIN].set(xt)
    xtp8 = xtp128[:, :8]

    idx_t = _knn(xtp8)
    idxf = idx_t.T.reshape(-1)

    agg0 = _gmax(xtp128, idxf, C)
    w0a = jnp.zeros((8, C), F32).at[:CIN, :].set(W0[:, :CIN].T)
    w0b = jnp.zeros((C, C), F32).at[:CIN, :].set(W0[:, CIN:].T)
    feat = _conv_in(xtp8, agg0, w0a, w0b, b0[None, :])

    for i in range(DEPTH):
        aggA = _gmax(feat, idxf, C)
        h = _conv_norm(feat, aggA, Wa[i, :, :C].T, Wa[i, :, C:].T,
                       ba[i][None, :], ga[i][None, :], bta[i][None, :], None)
        aggB = _gmax(h, idxf, C)
        feat = _conv_norm(h, aggB, Wb[i, :, :C].T, Wb[i, :, C:].T,
                          bb[i][None, :], gb[i][None, :], btb[i][None, :], feat)

    return jnp.transpose(feat[:N])[None]


# speedup vs baseline: 1.1620x; 1.1620x over previous
import functools

import jax
import jax.numpy as jnp
from jax import lax
from jax.experimental import pallas as pl
from jax.experimental.pallas import tpu as pltpu
from jax.experimental.pallas import tpu_sc as plsc

N = 10000
NP = 10240
CIN = 6
C = 128
K = 16
DEPTH = 6

BQ = 128
RC = 512
NCH = NP // RC
NW = 32
RPW = NP // NW
CH = 8
CF = 16

F32 = jnp.float32


NG = NP // 128
GK = 4


def _knn_body(xtp_ref, xqt_ref, idx_ref):
    xtp = xtp_ref[...]
    xqt = xqt_ref[...]
    prod = jnp.dot(xtp, xqt, preferred_element_type=F32)
    sqj = jnp.sum(xtp * xtp, axis=1, keepdims=True)
    sqq = jnp.sum(xqt * xqt, axis=0, keepdims=True)
    rowi = lax.broadcasted_iota(jnp.int32, (NP, 1), 0)
    d = (sqq + sqj) - 2.0 * prod
    d = jnp.where(rowi < N, d, jnp.float32(jnp.inf))

    riota = lax.broadcasted_iota(jnp.int32, (128, BQ), 0)
    gvs = [[] for _ in range(GK)]
    gas = [[] for _ in range(GK)]
    for g in range(NG):
        chunk = d[g * 128:(g + 1) * 128]
        for t in range(GK):
            m = jnp.min(chunk, axis=0, keepdims=True)
            a = jnp.min(jnp.where(chunk == m, riota, 128), axis=0,
                        keepdims=True)
            gvs[t].append(m)
            gas[t].append(a)
            if t + 1 < GK:
                chunk = jnp.where(riota == a, jnp.float32(jnp.inf), chunk)
    gv = [jnp.concatenate(v, axis=0) for v in gvs]
    ga = [jnp.concatenate(a, axis=0) for a in gas]

    giota = lax.broadcasted_iota(jnp.int32, (NG, BQ), 0)
    inf = jnp.float32(jnp.inf)
    hv, ha, hc = gv[0], ga[0], jnp.zeros((NG, BQ), jnp.int32)
    picks = []
    for _ in range(K):
        m = jnp.min(hv, axis=0, keepdims=True)
        s = jnp.min(jnp.where(hv == m, giota, NG), axis=0, keepdims=True)
        hit = giota == s
        j_in = jnp.min(jnp.where(hit, ha, NP), axis=0, keepdims=True)
        picks.append(s * 128 + j_in)
        hc = hc + hit.astype(jnp.int32)
        nv, na = inf, jnp.int32(NP)
        for t in range(GK - 1, 0, -1):
            sel = hc == t
            nv = jnp.where(sel, gv[t], nv)
            na = jnp.where(sel, ga[t], na)
        hv = jnp.where(hit, nv, hv)
        ha = jnp.where(hit, na, ha)
    idx_ref[...] = jnp.concatenate(picks, axis=0)


def _knn(xtp8):
    xtt = xtp8.T
    return pl.pallas_call(
        _knn_body,
        grid=(NP // BQ,),
        in_specs=[
            pl.BlockSpec((NP, 8), lambda i: (0, 0)),
            pl.BlockSpec((8, BQ), lambda i: (0, i)),
        ],
        out_specs=pl.BlockSpec((K, BQ), lambda i: (0, i)),
        out_shape=jax.ShapeDtypeStruct((K, NP), jnp.int32),
    )(xtp8, xtt)


NCHK = RPW // CF


def _gmax_body(D, feat_hbm, idx_hbm, out_hbm,
               idxall, rows0, rows1, outv0, outv1,
               gsem0, gsem1, osem0, osem1):
    wid = lax.axis_index("s") * 2 + lax.axis_index("c")
    row0 = wid * RPW
    rows = (rows0, rows1)
    outv = (outv0, outv1)
    gsem = (gsem0, gsem1)
    osem = (osem0, osem1)

    pltpu.sync_copy(idx_hbm.at[pl.ds(pl.multiple_of(row0 * K, 128), RPW * K)],
                    idxall)

    def fetch(i, buf):
        idxs = idxall.at[pl.ds(pl.multiple_of(i * CF * K, 128), CF * K)]
        pltpu.async_copy(feat_hbm.at[idxs], rows[buf], gsem[buf])

    def consume(i, buf, drain):
        idxs = idxall.at[pl.ds(pl.multiple_of(i * CF * K, 128), CF * K)]
        pltpu.make_async_copy(feat_hbm.at[idxs], rows[buf], gsem[buf]).wait()
        base = row0 + i * CF
        dst = out_hbm.at[pl.ds(pl.multiple_of(base, 8), CF)]

        @pl.when(drain)
        def _():
            pltpu.make_async_copy(outv[buf], dst, osem[buf]).wait()

        r = rows[buf]

        def sub(s, carry):
            r0 = s * CH * K
            for dd in range(CH):
                for lc in range(D // 16):
                    sl = pl.ds(lc * 16, 16)
                    acc = r[r0 + dd * K, sl]
                    for j in range(1, K):
                        acc = jnp.maximum(acc, r[r0 + dd * K + j, sl])
                    outv[buf][s * CH + dd, sl] = acc
            return carry

        lax.fori_loop(0, CF // CH, sub, 0)
        pltpu.async_copy(outv[buf], dst, osem[buf])

    fetch(0, 0)

    def pair(i2, carry):
        e = i2 * 2
        fetch(e + 1, 1)
        consume(e, 0, i2 > 0)

        @pl.when(e + 2 < NCHK)
        def _():
            fetch(e + 2, 0)

        consume(e + 1, 1, i2 > 0)
        return carry

    lax.fori_loop(0, NCHK // 2, pair, 0)
    tail = out_hbm.at[pl.ds(pl.multiple_of(row0, 8), CF)]
    pltpu.make_async_copy(outv[0], tail, osem[0]).wait()
    pltpu.make_async_copy(outv[1], tail, osem[1]).wait()


def _gmax(feat, idxf, D):
    mesh = plsc.VectorSubcoreMesh(core_axis_name="c", subcore_axis_name="s")
    return pl.kernel(
        functools.partial(_gmax_body, D),
        out_type=jax.ShapeDtypeStruct((NP, D), F32),
        mesh=mesh,
        scratch_types=[
            pltpu.VMEM((RPW * K,), jnp.int32),
            pltpu.VMEM((CF * K, D), F32),
            pltpu.VMEM((CF * K, D), F32),
            pltpu.VMEM((CF, D), F32),
            pltpu.VMEM((CF, D), F32),
            pltpu.SemaphoreType.DMA,
            pltpu.SemaphoreType.DMA,
            pltpu.SemaphoreType.DMA,
            pltpu.SemaphoreType.DMA,
        ],
    )(feat, idxf)


def _conv_in_body(f_ref, a_ref, w1_ref, w2_ref, b_ref, o_ref):
    o_ref[...] = (jnp.dot(f_ref[...], w1_ref[...], preferred_element_type=F32)
                  + jnp.dot(a_ref[...], w2_ref[...], preferred_element_type=F32)
                  + b_ref[...])


def _conv_in(xtp8, agg0, w1, w2, b):
    return pl.pallas_call(
        _conv_in_body,
        grid=(NCH,),
        in_specs=[
            pl.BlockSpec((RC, 8), lambda c: (c, 0)),
            pl.BlockSpec((RC, C), lambda c: (c, 0)),
            pl.BlockSpec((8, C), lambda c: (0, 0)),
            pl.BlockSpec((C, C), lambda c: (0, 0)),
            pl.BlockSpec((1, C), lambda c: (0, 0)),
        ],
        out_specs=pl.BlockSpec((RC, C), lambda c: (c, 0)),
        out_shape=jax.ShapeDtypeStruct((NP, C), F32),
    )(xtp8, agg0, w1, w2, b)


def _conv_norm_body(with_res, *refs):
    if with_res:
        f_ref, a_ref, w1_ref, w2_ref, b_ref, g_ref, bt_ref, r_ref, o_ref, acc = refs
    else:
        f_ref, a_ref, w1_ref, w2_ref, b_ref, g_ref, bt_ref, o_ref, acc = refs
    p = pl.program_id(0)
    c = pl.program_id(1)
    y = (jnp.dot(f_ref[...], w1_ref[...], preferred_element_type=F32)
         + jnp.dot(a_ref[...], w2_ref[...], preferred_element_type=F32)
         + b_ref[...])
    rows = lax.broadcasted_iota(jnp.int32, (RC, 1), 0) + c * RC
    msk = rows < N

    @pl.when(p == 0)
    def _():
        @pl.when(c == 0)
        def _():
            acc[...] = jnp.zeros_like(acc)
        ym = jnp.where(msk, y, 0.0)
        acc[0:1, :] = acc[0:1, :] + jnp.sum(ym, axis=0, keepdims=True)
        acc[1:2, :] = acc[1:2, :] + jnp.sum(ym * ym, axis=0, keepdims=True)

    @pl.when(p == 1)
    def _():
        m = acc[0:1, :] * (1.0 / N)
        v = acc[1:2, :] * (1.0 / N) - m * m
        s1 = 1.0 / jnp.sqrt(v + 1e-3)
        v2 = v * s1 * s1
        s = g_ref[...] * s1 / jnp.sqrt(v2 + 1e-5)
        out = jnp.maximum((y - m) * s + bt_ref[...], 0.0)
        if with_res:
            out = out + r_ref[...]
        o_ref[...] = out


def _conv_norm(feat, agg, w1, w2, b, g, bt, res):
    with_res = res is not None
    ops = (feat, agg, w1, w2, b, g, bt) + ((res,) if with_res else ())
    chunked = pl.BlockSpec((RC, C), lambda p, c: (c, 0))
    whole = lambda shape: pl.BlockSpec(shape, lambda p, c: (0, 0))
    in_specs = [chunked, chunked, whole((C, C)), whole((C, C)), whole((1, C)),
                whole((1, C)), whole((1, C))] + ([chunked] if with_res else [])
    return pl.pallas_call(
        functools.partial(_conv_norm_body, with_res),
        grid=(2, NCH),
        in_specs=in_specs,
        out_specs=chunked,
        out_shape=jax.ShapeDtypeStruct((NP, C), F32),
        scratch_shapes=[pltpu.VMEM((8, C), F32)],
    )(*ops)


def kernel(x, W0, b0, Wa, ba, ga, bta, Wb, bb, gb, btb):
    xt = x[0].T
    xtp128 = jnp.zeros((NP, C), F32).at[:N, :CIN].set(xt)
    xtp8 = xtp128[:, :8]

    idx_t = _knn(xtp8)
    idxf = idx_t.T.reshape(-1)

    agg0 = _gmax(xtp128, idxf, C)
    w0a = jnp.zeros((8, C), F32).at[:CIN, :].set(W0[:, :CIN].T)
    w0b = jnp.zeros((C, C), F32).at[:CIN, :].set(W0[:, CIN:].T)
    feat = _conv_in(xtp8, agg0, w0a, w0b, b0[None, :])

    for i in range(DEPTH):
        aggA = _gmax(feat, idxf, C)
        h = _conv_norm(feat, aggA, Wa[i, :, :C].T, Wa[i, :, C:].T,
                       ba[i][None, :], ga[i][None, :], bta[i][None, :], None)
        aggB = _gmax(h, idxf, C)
        feat = _conv_norm(h, aggB, Wb[i, :, :C].T, Wb[i, :, C:].T,
                          bb[i][None, :], gb[i][None, :], btb[i][None, :], feat)

    return jnp.transpose(feat[:N])[None]


# speedup vs baseline: 1.4751x; 1.2695x over previous
import functools

import jax
import jax.numpy as jnp
from jax import lax
from jax.experimental import pallas as pl
from jax.experimental.pallas import tpu as pltpu
from jax.experimental.pallas import tpu_sc as plsc

N = 10000
NP = 10240
CIN = 6
C = 128
K = 16
DEPTH = 6

BQ = 128
RC = 512
NCH = NP // RC
NW = 32
RPW = NP // NW
CH = 4
CF = 8

F32 = jnp.float32


NG = NP // 128
GK = 4


def _knn_body(xtp_ref, xqt_ref, idx_ref):
    xtp = xtp_ref[...]
    xqt = xqt_ref[...]
    prod = jnp.dot(xtp, xqt, preferred_element_type=F32)
    sqj = jnp.sum(xtp * xtp, axis=1, keepdims=True)
    sqq = jnp.sum(xqt * xqt, axis=0, keepdims=True)
    rowi = lax.broadcasted_iota(jnp.int32, (NP, 1), 0)
    d = (sqq + sqj) - 2.0 * prod
    d = jnp.where(rowi < N, d, jnp.float32(jnp.inf))

    riota = lax.broadcasted_iota(jnp.int32, (128, BQ), 0)
    gvs = [[] for _ in range(GK)]
    gas = [[] for _ in range(GK)]
    for g in range(NG):
        chunk = d[g * 128:(g + 1) * 128]
        for t in range(GK):
            m = jnp.min(chunk, axis=0, keepdims=True)
            a = jnp.min(jnp.where(chunk == m, riota, 128), axis=0,
                        keepdims=True)
            gvs[t].append(m)
            gas[t].append(a)
            if t + 1 < GK:
                chunk = jnp.where(riota == a, jnp.float32(jnp.inf), chunk)
    gv = [jnp.concatenate(v, axis=0) for v in gvs]
    ga = [jnp.concatenate(a, axis=0) for a in gas]

    giota = lax.broadcasted_iota(jnp.int32, (NG, BQ), 0)
    inf = jnp.float32(jnp.inf)
    hv, ha, hc = gv[0], ga[0], jnp.zeros((NG, BQ), jnp.int32)
    picks = []
    for _ in range(K):
        m = jnp.min(hv, axis=0, keepdims=True)
        s = jnp.min(jnp.where(hv == m, giota, NG), axis=0, keepdims=True)
        hit = giota == s
        j_in = jnp.min(jnp.where(hit, ha, NP), axis=0, keepdims=True)
        picks.append(s * 128 + j_in)
        hc = hc + hit.astype(jnp.int32)
        nv, na = inf, jnp.int32(NP)
        for t in range(GK - 1, 0, -1):
            sel = hc == t
            nv = jnp.where(sel, gv[t], nv)
            na = jnp.where(sel, ga[t], na)
        hv = jnp.where(hit, nv, hv)
        ha = jnp.where(hit, na, ha)
    idx_ref[...] = jnp.concatenate(picks, axis=0)


def _knn(xtp8):
    xtt = xtp8.T
    return pl.pallas_call(
        _knn_body,
        grid=(NP // BQ,),
        in_specs=[
            pl.BlockSpec((NP, 8), lambda i: (0, 0)),
            pl.BlockSpec((8, BQ), lambda i: (0, i)),
        ],
        out_specs=pl.BlockSpec((K, BQ), lambda i: (0, i)),
        out_shape=jax.ShapeDtypeStruct((K, NP), jnp.int32),
    )(xtp8, xtt)


NCHK = RPW // CF


def _gmax_body(D, feat_hbm, idx_hbm, out_hbm,
               idxall, rows0, rows1, rows2, rows3,
               outv0, outv1, outv2, outv3,
               gsem0, gsem1, gsem2, gsem3,
               osem0, osem1, osem2, osem3):
    wid = lax.axis_index("s") * 2 + lax.axis_index("c")
    row0 = wid * RPW
    rows = (rows0, rows1, rows2, rows3)
    outv = (outv0, outv1, outv2, outv3)
    gsem = (gsem0, gsem1, gsem2, gsem3)
    osem = (osem0, osem1, osem2, osem3)

    pltpu.sync_copy(idx_hbm.at[pl.ds(pl.multiple_of(row0 * K, 128), RPW * K)],
                    idxall)

    def fetch(i, buf):
        idxs = idxall.at[pl.ds(pl.multiple_of(i * CF * K, 128), CF * K)]
        pltpu.async_copy(feat_hbm.at[idxs], rows[buf], gsem[buf])

    def consume(i, buf, drain):
        idxs = idxall.at[pl.ds(pl.multiple_of(i * CF * K, 128), CF * K)]
        pltpu.make_async_copy(feat_hbm.at[idxs], rows[buf], gsem[buf]).wait()
        base = row0 + i * CF
        dst = out_hbm.at[pl.ds(pl.multiple_of(base, 8), CF)]

        @pl.when(drain)
        def _():
            pltpu.make_async_copy(outv[buf], dst, osem[buf]).wait()

        r = rows[buf]

        def sub(s, carry):
            r0 = s * CH * K
            for dd in range(CH):
                for lc in range(D // 16):
                    sl = pl.ds(lc * 16, 16)
                    acc = r[r0 + dd * K, sl]
                    for j in range(1, K):
                        acc = jnp.maximum(acc, r[r0 + dd * K + j, sl])
                    outv[buf][s * CH + dd, sl] = acc
            return carry

        lax.fori_loop(0, CF // CH, sub, 0)
        pltpu.async_copy(outv[buf], dst, osem[buf])

    fetch(0, 0)
    fetch(1, 1)
    fetch(2, 2)

    def quad(q, carry):
        e = q * 4
        for b in range(4):
            i = e + b

            @pl.when(i + 3 < NCHK)
            def _():
                fetch(i + 3, (b + 3) % 4)

            consume(i, b, q > 0)
        return carry

    lax.fori_loop(0, NCHK // 4, quad, 0)
    tail = out_hbm.at[pl.ds(pl.multiple_of(row0, 8), CF)]
    for b in range(4):
        pltpu.make_async_copy(outv[b], tail, osem[b]).wait()


def _gmax(feat, idxf, D):
    mesh = plsc.VectorSubcoreMesh(core_axis_name="c", subcore_axis_name="s")
    return pl.kernel(
        functools.partial(_gmax_body, D),
        out_type=jax.ShapeDtypeStruct((NP, D), F32),
        mesh=mesh,
        scratch_types=[
            pltpu.VMEM((RPW * K,), jnp.int32),
            pltpu.VMEM((CF * K, D), F32),
            pltpu.VMEM((CF * K, D), F32),
            pltpu.VMEM((CF * K, D), F32),
            pltpu.VMEM((CF * K, D), F32),
            pltpu.VMEM((CF, D), F32),
            pltpu.VMEM((CF, D), F32),
            pltpu.VMEM((CF, D), F32),
            pltpu.VMEM((CF, D), F32),
            pltpu.SemaphoreType.DMA,
            pltpu.SemaphoreType.DMA,
            pltpu.SemaphoreType.DMA,
            pltpu.SemaphoreType.DMA,
            pltpu.SemaphoreType.DMA,
            pltpu.SemaphoreType.DMA,
            pltpu.SemaphoreType.DMA,
            pltpu.SemaphoreType.DMA,
        ],
    )(feat, idxf)


def _conv_in_body(f_ref, a_ref, w1_ref, w2_ref, b_ref, o_ref):
    o_ref[...] = (jnp.dot(f_ref[...], w1_ref[...], preferred_element_type=F32)
                  + jnp.dot(a_ref[...], w2_ref[...], preferred_element_type=F32)
                  + b_ref[...])


def _conv_in(xtp8, agg0, w1, w2, b):
    return pl.pallas_call(
        _conv_in_body,
        grid=(NCH,),
        in_specs=[
            pl.BlockSpec((RC, 8), lambda c: (c, 0)),
            pl.BlockSpec((RC, C), lambda c: (c, 0)),
            pl.BlockSpec((8, C), lambda c: (0, 0)),
            pl.BlockSpec((C, C), lambda c: (0, 0)),
            pl.BlockSpec((1, C), lambda c: (0, 0)),
        ],
        out_specs=pl.BlockSpec((RC, C), lambda c: (c, 0)),
        out_shape=jax.ShapeDtypeStruct((NP, C), F32),
    )(xtp8, agg0, w1, w2, b)


def _conv_norm_body(with_res, *refs):
    if with_res:
        f_ref, a_ref, w1_ref, w2_ref, b_ref, g_ref, bt_ref, r_ref, o_ref, acc = refs
    else:
        f_ref, a_ref, w1_ref, w2_ref, b_ref, g_ref, bt_ref, o_ref, acc = refs
    p = pl.program_id(0)
    c = pl.program_id(1)
    y = (jnp.dot(f_ref[...], w1_ref[...], preferred_element_type=F32)
         + jnp.dot(a_ref[...], w2_ref[...], preferred_element_type=F32)
         + b_ref[...])
    rows = lax.broadcasted_iota(jnp.int32, (RC, 1), 0) + c * RC
    msk = rows < N

    @pl.when(p == 0)
    def _():
        @pl.when(c == 0)
        def _():
            acc[...] = jnp.zeros_like(acc)
        ym = jnp.where(msk, y, 0.0)
        acc[0:1, :] = acc[0:1, :] + jnp.sum(ym, axis=0, keepdims=True)
        acc[1:2, :] = acc[1:2, :] + jnp.sum(ym * ym, axis=0, keepdims=True)

    @pl.when(p == 1)
    def _():
        m = acc[0:1, :] * (1.0 / N)
        v = acc[1:2, :] * (1.0 / N) - m * m
        s1 = 1.0 / jnp.sqrt(v + 1e-3)
        v2 = v * s1 * s1
        s = g_ref[...] * s1 / jnp.sqrt(v2 + 1e-5)
        out = jnp.maximum((y - m) * s + bt_ref[...], 0.0)
        if with_res:
            out = out + r_ref[...]
        o_ref[...] = out


def _conv_norm(feat, agg, w1, w2, b, g, bt, res):
    with_res = res is not None
    ops = (feat, agg, w1, w2, b, g, bt) + ((res,) if with_res else ())
    chunked = pl.BlockSpec((RC, C), lambda p, c: (c, 0))
    whole = lambda shape: pl.BlockSpec(shape, lambda p, c: (0, 0))
    in_specs = [chunked, chunked, whole((C, C)), whole((C, C)), whole((1, C)),
                whole((1, C)), whole((1, C))] + ([chunked] if with_res else [])
    return pl.pallas_call(
        functools.partial(_conv_norm_body, with_res),
        grid=(2, NCH),
        in_specs=in_specs,
        out_specs=chunked,
        out_shape=jax.ShapeDtypeStruct((NP, C), F32),
        scratch_shapes=[pltpu.VMEM((8, C), F32)],
    )(*ops)


def kernel(x, W0, b0, Wa, ba, ga, bta, Wb, bb, gb, btb):
    xt = x[0].T
    xtp128 = jnp.zeros((NP, C), F32).at[:N, :CIN].set(xt)
    xtp8 = xtp128[:, :8]

    idx_t = _knn(xtp8)
    idxf = idx_t.T.reshape(-1)

    agg0 = _gmax(xtp128, idxf, C)
    w0a = jnp.zeros((8, C), F32).at[:CIN, :].set(W0[:, :CIN].T)
    w0b = jnp.zeros((C, C), F32).at[:CIN, :].set(W0[:, CIN:].T)
    feat = _conv_in(xtp8, agg0, w0a, w0b, b0[None, :])

    for i in range(DEPTH):
        aggA = _gmax(feat, idxf, C)
        h = _conv_norm(feat, aggA, Wa[i, :, :C].T, Wa[i, :, C:].T,
                       ba[i][None, :], ga[i][None, :], bta[i][None, :], None)
        aggB = _gmax(h, idxf, C)
        feat = _conv_norm(h, aggB, Wb[i, :, :C].T, Wb[i, :, C:].T,
                          bb[i][None, :], gb[i][None, :], btb[i][None, :], feat)

    return jnp.transpose(feat[:N])[None]
